# baseline (speedup 1.0000x reference)
import functools

import jax
import jax.numpy as jnp
from jax import lax
from jax.experimental import pallas as pl
from jax.experimental.pallas import tpu as pltpu
from jax.experimental.pallas import tpu_sc as plsc

N_CODES = 8192
D_EMB = 32
COMMIT_BETA = 0.25

RB1 = 256
CB1 = 4096
OH_CHUNK = 2048

SC_CORES = 2
SC_SUBCORES = 16
SC_WORKERS = SC_CORES * SC_SUBCORES


def _vq_body(nr, zsq_ref, z_ref, wt_ref, wsq_ref,
             idx_ref, enc_ref, cnt_ref, best_ref, bidx_ref, pidx_ref):
    r = pl.program_id(0)
    ct = pl.program_id(1)

    @pl.when(r < nr)
    def _():
        zb = z_ref[...].astype(jnp.bfloat16)
        wb2 = wt_ref[...].astype(jnp.bfloat16) * jnp.bfloat16(2.0)
        mm2 = lax.dot_general(
            zb, wb2, (((1,), (0,)), ((), ())),
            preferred_element_type=jnp.float32)
        d = (zsq_ref[...] + wsq_ref[...]) - mm2
        m = jnp.min(d, axis=1, keepdims=True)
        cols = lax.broadcasted_iota(jnp.int32, d.shape, 1) + ct * CB1
        cand = jnp.where(d == m, cols, jnp.int32(2 ** 30))
        am = jnp.min(cand, axis=1, keepdims=True)

        @pl.when(ct == 0)
        def _():
            best_ref[...] = m
            bidx_ref[...] = am

        @pl.when(ct == 1)
        def _():
            m0q = best_ref[...].astype(jnp.bfloat16).astype(jnp.float32)
            comb = jnp.where(m < m0q, am, bidx_ref[...])
            idx_ref[...] = comb.astype(jnp.int32)

    @pl.when(ct == 1)
    def _():
        @pl.when(r == 1)
        def _():
            cnt_ref[...] = jnp.zeros_like(cnt_ref)

        @pl.when(r > 0)
        def _():
            pid = pidx_ref[...]
            for k in range(N_CODES // OH_CHUNK):
                sl = slice(k * OH_CHUNK, (k + 1) * OH_CHUNK)
                cols = (lax.broadcasted_iota(jnp.int32, (RB1, OH_CHUNK), 1)
                        + k * OH_CHUNK)
                oh = (pid == cols).astype(jnp.float32)
                enc_ref[:, sl] = oh
                cnt_ref[0:1, sl] += jnp.sum(oh, axis=0, keepdims=True)

        @pl.when(r < nr)
        def _():
            pidx_ref[...] = idx_ref[...]


def _vq_call(zsq, zf, wt, wsq):
    n = zf.shape[0]
    nr = n // RB1
    grid = (nr + 1, 2)
    return pl.pallas_call(
        functools.partial(_vq_body, nr),
        grid=grid,
        in_specs=[
            pl.BlockSpec((RB1, 1), lambda rt, ct: (jnp.minimum(rt, nr - 1), 0)),
            pl.BlockSpec((RB1, D_EMB), lambda rt, ct: (jnp.minimum(rt, nr - 1), 0)),
            pl.BlockSpec((D_EMB, CB1), lambda rt, ct: (0, ct)),
            pl.BlockSpec((1, CB1), lambda rt, ct: (0, ct)),
        ],
        out_specs=[
            pl.BlockSpec((RB1, 1), lambda rt, ct: (jnp.minimum(rt, nr - 1), 0)),
            pl.BlockSpec((RB1, N_CODES), lambda rt, ct: (jnp.maximum(rt, 1) - 1, 0)),
            pl.BlockSpec((1, N_CODES), lambda rt, ct: (0, 0)),
        ],
        out_shape=[
            jax.ShapeDtypeStruct((n, 1), jnp.int32),
            jax.ShapeDtypeStruct((n, N_CODES), jnp.float32),
            jax.ShapeDtypeStruct((1, N_CODES), jnp.float32),
        ],
        scratch_shapes=[
            pltpu.VMEM((RB1, 1), jnp.float32),
            pltpu.VMEM((RB1, 1), jnp.int32),
            pltpu.VMEM((RB1, 1), jnp.int32),
        ],
    )(zsq, zf, wt, wsq)


def _make_gather(n):
    rows_per_w = n // SC_WORKERS
    idx_rows = rows_per_w // 128
    mesh = plsc.VectorSubcoreMesh(core_axis_name="c", subcore_axis_name="s")

    @functools.partial(
        pl.kernel,
        mesh=mesh,
        compiler_params=pltpu.CompilerParams(use_tc_tiling_on_sc=False),
        out_type=jax.ShapeDtypeStruct((n, D_EMB), jnp.float32),
        scratch_types=[
            pltpu.VMEM((idx_rows, 128), jnp.int32),
            pltpu.VMEM((rows_per_w, D_EMB), jnp.float32),
            pltpu.SemaphoreType.DMA,
        ],
    )
    def _gather(w_hbm, idx_hbm, out_hbm, idx_v, rows_v, sem):
        wid = lax.axis_index("s") * SC_CORES + lax.axis_index("c")
        pltpu.sync_copy(idx_hbm.at[pl.ds(wid * idx_rows, idx_rows)], idx_v)
        copies = []
        for j in range(idx_rows):
            copies.append(pltpu.async_copy(
                w_hbm.at[idx_v.at[j]],
                rows_v.at[pl.ds(j * 128, 128)],
                sem))
        for c in copies:
            c.wait()
        pltpu.sync_copy(rows_v, out_hbm.at[pl.ds(wid * rows_per_w, rows_per_w)])

    return _gather


def kernel(z, W):
    batch, seq, d_emb = z.shape
    zf = z.reshape(-1, d_emb)
    n = zf.shape[0]

    zsq = jnp.sum(z ** 2, axis=2).reshape(-1, 1)
    wsq = jnp.sum(W ** 2, axis=1)

    idx2, enc, cnt = _vq_call(zsq, zf, W.T, wsq.reshape(1, -1))

    idx_flat = idx2[:, 0]
    zq_flat = _make_gather(n)(W, idx_flat.reshape(-1, 128))
    z_q = zq_flat.astype(jnp.bfloat16).astype(jnp.float32).reshape(batch, seq, d_emb)

    loss = (jnp.mean((lax.stop_gradient(z_q) - z) ** 2)
            + COMMIT_BETA * jnp.mean((z_q - lax.stop_gradient(z)) ** 2))
    z_q_st = z + lax.stop_gradient(z_q - z)
    e_mean = cnt[0] / jnp.float32(n)
    perplexity = jnp.exp(-jnp.sum(e_mean * jnp.log(e_mean + 1e-10)))
    idx_out = idx_flat.reshape(batch, seq)
    return (loss, z_q_st, perplexity, enc, idx_out)


# speedup vs baseline: 1.5835x; 1.5835x over previous
import functools

import jax
import jax.numpy as jnp
from jax import lax
from jax.experimental import pallas as pl
from jax.experimental.pallas import tpu as pltpu
from jax.experimental.pallas import tpu_sc as plsc

N_CODES = 8192
D_EMB = 32
COMMIT_BETA = 0.25

RB1 = 256
CB1 = 4096
OH_CHUNK = 2048

SC_CORES = 2
SC_SUBCORES = 16
SC_WORKERS = SC_CORES * SC_SUBCORES


def _vq_body(nr, zsq_ref, z_ref, wt_ref, wsq_ref,
             idx_ref, enc_ref, cnt_ref, pidx_ref):
    r = pl.program_id(0)

    @pl.when(r < nr)
    def _():
        zb = z_ref[...].astype(jnp.bfloat16)
        wb2 = wt_ref[...].astype(jnp.bfloat16) * jnp.bfloat16(2.0)
        mm2 = lax.dot_general(
            zb, wb2, (((1,), (0,)), ((), ())),
            preferred_element_type=jnp.float32)
        zs = zsq_ref[...]
        wsq = wsq_ref[...]

        lane = lax.broadcasted_iota(jnp.int32, (RB1, 128), 1)

        def half_argmin(base):
            def strip(s):
                sl = slice(base + s * 128, base + (s + 1) * 128)
                return (zs + wsq[:, sl]) - mm2[:, sl]

            v = strip(0)
            st = jnp.zeros((RB1, 128), jnp.int32)
            for s in range(1, CB1 // 128):
                dv = strip(s)
                take = dv < v
                v = jnp.where(take, dv, v)
                st = jnp.where(take, jnp.int32(s), st)
            m = jnp.min(v, axis=1, keepdims=True)
            cand = jnp.where(v == m, st * 128 + lane, jnp.int32(2 ** 30))
            return m, jnp.min(cand, axis=1, keepdims=True)

        m0, am0 = half_argmin(0)
        m1, am1 = half_argmin(CB1)
        m0q = m0.astype(jnp.bfloat16).astype(jnp.float32)
        idx_ref[...] = jnp.where(m1 < m0q, am1 + CB1, am0).astype(jnp.int32)

    @pl.when(r == 1)
    def _():
        cnt_ref[...] = jnp.zeros_like(cnt_ref)

    @pl.when(r > 0)
    def _():
        pid = pidx_ref[...]
        for k in range(N_CODES // OH_CHUNK):
            sl = slice(k * OH_CHUNK, (k + 1) * OH_CHUNK)
            ccols = (lax.broadcasted_iota(jnp.int32, (RB1, OH_CHUNK), 1)
                     + k * OH_CHUNK)
            oh = (pid == ccols).astype(jnp.float32)
            enc_ref[:, sl] = oh
            cnt_ref[0:1, sl] += jnp.sum(oh, axis=0, keepdims=True)

    @pl.when(r < nr)
    def _():
        pidx_ref[...] = idx_ref[...]


def _vq_call(zsq, zf, wt, wsq):
    n = zf.shape[0]
    nr = n // RB1
    grid = (nr + 1,)
    return pl.pallas_call(
        functools.partial(_vq_body, nr),
        grid=grid,
        in_specs=[
            pl.BlockSpec((RB1, 1), lambda rt: (jnp.minimum(rt, nr - 1), 0)),
            pl.BlockSpec((RB1, D_EMB), lambda rt: (jnp.minimum(rt, nr - 1), 0)),
            pl.BlockSpec((D_EMB, N_CODES), lambda rt: (0, 0)),
            pl.BlockSpec((1, N_CODES), lambda rt: (0, 0)),
        ],
        out_specs=[
            pl.BlockSpec((RB1, 1), lambda rt: (jnp.minimum(rt, nr - 1), 0)),
            pl.BlockSpec((RB1, N_CODES), lambda rt: (jnp.maximum(rt, 1) - 1, 0)),
            pl.BlockSpec((1, N_CODES), lambda rt: (0, 0)),
        ],
        out_shape=[
            jax.ShapeDtypeStruct((n, 1), jnp.int32),
            jax.ShapeDtypeStruct((n, N_CODES), jnp.float32),
            jax.ShapeDtypeStruct((1, N_CODES), jnp.float32),
        ],
        scratch_shapes=[
            pltpu.VMEM((RB1, 1), jnp.int32),
        ],
    )(zsq, zf, wt, wsq)


def _make_gather(n):
    rows_per_w = n // SC_WORKERS
    idx_rows = rows_per_w // 128
    mesh = plsc.VectorSubcoreMesh(core_axis_name="c", subcore_axis_name="s")

    @functools.partial(
        pl.kernel,
        mesh=mesh,
        compiler_params=pltpu.CompilerParams(use_tc_tiling_on_sc=False),
        out_type=jax.ShapeDtypeStruct((n, D_EMB), jnp.float32),
        scratch_types=[
            pltpu.VMEM((idx_rows, 128), jnp.int32),
            pltpu.VMEM((rows_per_w, D_EMB), jnp.float32),
            pltpu.SemaphoreType.DMA,
        ],
    )
    def _gather(w_hbm, idx_hbm, out_hbm, idx_v, rows_v, sem):
        wid = lax.axis_index("s") * SC_CORES + lax.axis_index("c")
        pltpu.sync_copy(idx_hbm.at[pl.ds(wid * idx_rows, idx_rows)], idx_v)
        copies = []
        for j in range(idx_rows):
            copies.append(pltpu.async_copy(
                w_hbm.at[idx_v.at[j]],
                rows_v.at[pl.ds(j * 128, 128)],
                sem))
        for c in copies:
            c.wait()
        pltpu.sync_copy(rows_v, out_hbm.at[pl.ds(wid * rows_per_w, rows_per_w)])

    return _gather


def kernel(z, W):
    batch, seq, d_emb = z.shape
    zf = z.reshape(-1, d_emb)
    n = zf.shape[0]

    zsq = jnp.sum(z ** 2, axis=2).reshape(-1, 1)
    wsq = jnp.sum(W ** 2, axis=1)

    idx2, enc, cnt = _vq_call(zsq, zf, W.T, wsq.reshape(1, -1))

    idx_flat = idx2[:, 0]
    zq_flat = _make_gather(n)(W, idx_flat.reshape(-1, 128))
    z_q = zq_flat.astype(jnp.bfloat16).astype(jnp.float32).reshape(batch, seq, d_emb)

    loss = (jnp.mean((lax.stop_gradient(z_q) - z) ** 2)
            + COMMIT_BETA * jnp.mean((z_q - lax.stop_gradient(z)) ** 2))
    z_q_st = z + lax.stop_gradient(z_q - z)
    e_mean = cnt[0] / jnp.float32(n)
    perplexity = jnp.exp(-jnp.sum(e_mean * jnp.log(e_mean + 1e-10)))
    idx_out = idx_flat.reshape(batch, seq)
    return (loss, z_q_st, perplexity, enc, idx_out)
